# baseline (speedup 1.0000x reference)
import functools

import jax
import jax.numpy as jnp
from jax import lax
from jax.experimental import pallas as pl
from jax.experimental.pallas import tpu as pltpu
from jax.experimental.pallas import tpu_sc as plsc


def _sc_gather(table, flat_ids):
    bs = flat_ids.shape[0]
    d = table.shape[1]
    info = plsc.get_sparse_core_info()
    nw = info.num_cores * info.num_subcores
    per_w = bs // nw
    chunk = next(c for c in (80, 64, 32, 16, 8, 4, 2, 1) if per_w % c == 0)
    n_chunks = per_w // chunk
    mesh = plsc.VectorSubcoreMesh(core_axis_name="c", subcore_axis_name="s")

    @functools.partial(
        pl.kernel,
        mesh=mesh,
        out_type=jax.ShapeDtypeStruct((bs, d), jnp.float32),
        scratch_types=[
            pltpu.VMEM((per_w,), jnp.int32),
            pltpu.VMEM((chunk, d), jnp.float32),
            pltpu.VMEM((chunk, d), jnp.float32),
            pltpu.SemaphoreType.DMA,
            pltpu.SemaphoreType.DMA,
        ],
    )
    def k(ids_hbm, table_hbm, out_hbm, idx_v, rows0, rows1, sem0, sem1):
        wid = lax.axis_index("s") * info.num_cores + lax.axis_index("c")
        base = wid * per_w
        pltpu.sync_copy(ids_hbm.at[pl.ds(base, per_w)], idx_v)
        rows = (rows0, rows1)
        sems = (sem0, sem1)
        pltpu.async_copy(table_hbm.at[idx_v.at[pl.ds(0, chunk)]], rows[0], sems[0])
        for c in range(n_chunks):
            if c + 1 < n_chunks:
                pltpu.async_copy(
                    table_hbm.at[idx_v.at[pl.ds((c + 1) * chunk, chunk)]],
                    rows[(c + 1) % 2], sems[(c + 1) % 2])
            pltpu.make_async_copy(
                table_hbm.at[idx_v.at[pl.ds(c * chunk, chunk)]],
                rows[c % 2], sems[c % 2]).wait()
            pltpu.sync_copy(rows[c % 2], out_hbm.at[pl.ds(base + c * chunk, chunk)])

    return k(flat_ids, table)


def _tc_compute(g_ref, pos_ref, seg_ref, type_ref, gamma_ref, beta_ref,
                w_ref, b_ref, out_ref):
    t0 = type_ref[0, :][None, :]
    t1 = type_ref[1, :][None, :]
    sf = seg_ref[0, 0, :][:, None]
    emb = g_ref[...] + pos_ref[...].astype(jnp.float32) + (t0 + sf * (t1 - t0))
    mu = jnp.mean(emb, axis=1, keepdims=True)
    xc = emb - mu
    var = jnp.mean(xc * xc, axis=1, keepdims=True)
    y = xc * lax.rsqrt(var + 1e-12) * gamma_ref[0, :] + beta_ref[0, :]
    out_ref[...] = jnp.dot(y.astype(jnp.bfloat16), w_ref[...],
                           preferred_element_type=jnp.float32) + b_ref[0, :]


def _tc_body(g_ref, pos_ref, seg_ref, type_ref, gamma_ref, beta_ref,
             w_ref, b_ref, out_ref):
    _tc_compute(g_ref, pos_ref, seg_ref, type_ref, gamma_ref, beta_ref,
                w_ref, b_ref, out_ref)


def _tc_body_alias(g_ref, pos_ref, seg_ref, type_ref, gamma_ref, beta_ref,
                   w_ref, b_ref, prev_ref, out_ref):
    _tc_compute(g_ref, pos_ref, seg_ref, type_ref, gamma_ref, beta_ref,
                w_ref, b_ref, out_ref)


def _tc_fuse_slice(gathered, pos_emb, segf, type_emb, gamma, beta, w, b,
                   prev, base_seq, total_rows):
    bs_k, d = gathered.shape
    nseq_k, _, s = segf.shape
    ins = [gathered, pos_emb, segf, type_emb, gamma, beta, w, b]
    in_specs = [
        pl.BlockSpec((s, d), lambda i: (i, 0)),
        pl.BlockSpec((s, d), lambda i: (0, 0)),
        pl.BlockSpec((1, 1, s), lambda i: (i, 0, 0)),
        pl.BlockSpec((2, d), lambda i: (0, 0)),
        pl.BlockSpec((1, d), lambda i: (0, 0)),
        pl.BlockSpec((1, d), lambda i: (0, 0)),
        pl.BlockSpec((d, d), lambda i: (0, 0)),
        pl.BlockSpec((1, d), lambda i: (0, 0)),
    ]
    kwargs = {}
    body = _tc_body
    if prev is not None:
        ins.append(prev)
        in_specs.append(pl.BlockSpec(memory_space=pl.ANY))
        kwargs["input_output_aliases"] = {8: 0}
        body = _tc_body_alias
    return pl.pallas_call(
        body,
        grid=(nseq_k,),
        in_specs=in_specs,
        out_specs=pl.BlockSpec((s, d), lambda i, b=base_seq: (i + b, 0)),
        out_shape=jax.ShapeDtypeStruct((total_rows, d), jnp.float32),
        **kwargs,
    )(*ins)


def kernel(input_ids, segment_ids, word_emb, pos_emb, type_emb,
           ln_gamma, ln_beta, W_proj, b_proj):
    nb, s = input_ids.shape
    d = word_emb.shape[1]
    flat_ids = input_ids.reshape(-1)
    blk = 2 if nb % 2 == 0 else 1
    segf = segment_ids.astype(jnp.float32).reshape(nb // blk, 1, blk * s)
    pos2 = jnp.concatenate([pos_emb] * blk, axis=0).astype(jnp.bfloat16)
    gamma2 = ln_gamma.reshape(1, d)
    beta2 = ln_beta.reshape(1, d)
    w_bf = W_proj.astype(jnp.bfloat16)
    b2 = b_proj.reshape(1, d)

    if nb == 64:
        slice_seqs = [8, 16, 16, 16, 8]
    else:
        slice_seqs = [nb]
    out = None
    base = 0
    for sl in slice_seqs:
        ids_k = lax.slice(flat_ids, (base * s,), ((base + sl) * s,))
        g_k = _sc_gather(word_emb, ids_k)
        segf_k = lax.slice(segf, (base // blk, 0, 0),
                           ((base + sl) // blk, 1, blk * s))
        out = _tc_fuse_slice(g_k, pos2, segf_k, type_emb,
                             gamma2, beta2, w_bf, b2,
                             out, base // blk, nb * s)
        base += sl
    return out.reshape(nb, s, d)


# speedup vs baseline: 1.0238x; 1.0238x over previous
import functools

import jax
import jax.numpy as jnp
from jax import lax
from jax.experimental import pallas as pl
from jax.experimental.pallas import tpu as pltpu
from jax.experimental.pallas import tpu_sc as plsc


def _sc_gather(table, flat_ids):
    bs = flat_ids.shape[0]
    d = table.shape[1]
    info = plsc.get_sparse_core_info()
    nw = info.num_cores * info.num_subcores
    per_w = bs // nw
    chunk = next(c for c in (80, 64, 32, 16, 8, 4, 2, 1) if per_w % c == 0)
    n_chunks = per_w // chunk
    mesh = plsc.VectorSubcoreMesh(core_axis_name="c", subcore_axis_name="s")

    @functools.partial(
        pl.kernel,
        mesh=mesh,
        out_type=jax.ShapeDtypeStruct((bs, d), jnp.float32),
        scratch_types=[
            pltpu.VMEM((per_w,), jnp.int32),
            pltpu.VMEM((chunk, d), jnp.float32),
            pltpu.VMEM((chunk, d), jnp.float32),
            pltpu.SemaphoreType.DMA,
            pltpu.SemaphoreType.DMA,
        ],
    )
    def k(ids_hbm, table_hbm, out_hbm, idx_v, rows0, rows1, sem0, sem1):
        wid = lax.axis_index("s") * info.num_cores + lax.axis_index("c")
        base = wid * per_w
        pltpu.sync_copy(ids_hbm.at[pl.ds(base, per_w)], idx_v)
        rows = (rows0, rows1)
        sems = (sem0, sem1)
        pltpu.async_copy(table_hbm.at[idx_v.at[pl.ds(0, chunk)]], rows[0], sems[0])
        for c in range(n_chunks):
            if c + 1 < n_chunks:
                pltpu.async_copy(
                    table_hbm.at[idx_v.at[pl.ds((c + 1) * chunk, chunk)]],
                    rows[(c + 1) % 2], sems[(c + 1) % 2])
            pltpu.make_async_copy(
                table_hbm.at[idx_v.at[pl.ds(c * chunk, chunk)]],
                rows[c % 2], sems[c % 2]).wait()
            pltpu.sync_copy(rows[c % 2], out_hbm.at[pl.ds(base + c * chunk, chunk)])

    return k(flat_ids, table)


def _tc_compute(g_ref, pos_ref, seg_ref, type_ref, gamma_ref, beta_ref,
                w_ref, b_ref, out_ref):
    t0 = type_ref[0, :][None, :]
    t1 = type_ref[1, :][None, :]
    sf = seg_ref[0, 0, :][:, None]
    emb = g_ref[...] + pos_ref[...].astype(jnp.float32) + (t0 + sf * (t1 - t0))
    mu = jnp.mean(emb, axis=1, keepdims=True)
    xc = emb - mu
    var = jnp.mean(xc * xc, axis=1, keepdims=True)
    y = xc * lax.rsqrt(var + 1e-12) * gamma_ref[0, :] + beta_ref[0, :]
    out_ref[...] = jnp.dot(y.astype(jnp.bfloat16), w_ref[...],
                           preferred_element_type=jnp.float32) + b_ref[0, :]


def _tc_body(g_ref, pos_ref, seg_ref, type_ref, gamma_ref, beta_ref,
             w_ref, b_ref, out_ref):
    _tc_compute(g_ref, pos_ref, seg_ref, type_ref, gamma_ref, beta_ref,
                w_ref, b_ref, out_ref)


def _tc_body_alias(g_ref, pos_ref, seg_ref, type_ref, gamma_ref, beta_ref,
                   w_ref, b_ref, prev_ref, out_ref):
    _tc_compute(g_ref, pos_ref, seg_ref, type_ref, gamma_ref, beta_ref,
                w_ref, b_ref, out_ref)


def _tc_fuse_slice(gathered, pos_emb, segf, type_emb, gamma, beta, w, b,
                   prev, base_seq, total_rows):
    bs_k, d = gathered.shape
    nseq_k, _, s = segf.shape
    ins = [gathered, pos_emb, segf, type_emb, gamma, beta, w, b]
    in_specs = [
        pl.BlockSpec((s, d), lambda i: (i, 0)),
        pl.BlockSpec((s, d), lambda i: (0, 0)),
        pl.BlockSpec((1, 1, s), lambda i: (i, 0, 0)),
        pl.BlockSpec((2, d), lambda i: (0, 0)),
        pl.BlockSpec((1, d), lambda i: (0, 0)),
        pl.BlockSpec((1, d), lambda i: (0, 0)),
        pl.BlockSpec((d, d), lambda i: (0, 0)),
        pl.BlockSpec((1, d), lambda i: (0, 0)),
    ]
    kwargs = {}
    body = _tc_body
    if prev is not None:
        ins.append(prev)
        in_specs.append(pl.BlockSpec(memory_space=pl.ANY))
        kwargs["input_output_aliases"] = {8: 0}
        body = _tc_body_alias
    return pl.pallas_call(
        body,
        grid=(nseq_k,),
        in_specs=in_specs,
        out_specs=pl.BlockSpec((s, d), lambda i, b=base_seq: (i + b, 0)),
        out_shape=jax.ShapeDtypeStruct((total_rows, d), jnp.float32),
        **kwargs,
    )(*ins)


def kernel(input_ids, segment_ids, word_emb, pos_emb, type_emb,
           ln_gamma, ln_beta, W_proj, b_proj):
    nb, s = input_ids.shape
    d = word_emb.shape[1]
    flat_ids = input_ids.reshape(-1)
    blk = 2 if nb % 2 == 0 else 1
    segf = segment_ids.astype(jnp.float32).reshape(nb // blk, 1, blk * s)
    pos2 = jnp.concatenate([pos_emb] * blk, axis=0).astype(jnp.bfloat16)
    gamma2 = ln_gamma.reshape(1, d)
    beta2 = ln_beta.reshape(1, d)
    w_bf = W_proj.astype(jnp.bfloat16)
    b2 = b_proj.reshape(1, d)

    if nb == 64:
        slice_seqs = [8, 16, 20, 20]
    else:
        slice_seqs = [nb]
    out = None
    base = 0
    for sl in slice_seqs:
        ids_k = lax.slice(flat_ids, (base * s,), ((base + sl) * s,))
        g_k = _sc_gather(word_emb, ids_k)
        segf_k = lax.slice(segf, (base // blk, 0, 0),
                           ((base + sl) // blk, 1, blk * s))
        out = _tc_fuse_slice(g_k, pos2, segf_k, type_emb,
                             gamma2, beta2, w_bf, b2,
                             out, base // blk, nb * s)
        base += sl
    return out.reshape(nb, s, d)
